# Initial kernel scaffold; baseline (speedup 1.0000x reference)
import jax
import jax.numpy as jnp
from jax import lax
from jax.experimental import pallas as pl
from jax.experimental.pallas import tpu as pltpu
from jax.experimental.pallas import tpu_sc as plsc

_N = 10000
_D = 128
_E = 320000
_NC = 2
_NS = 16
_W = 125
_WPW = _E // (_NC * _NS * _W)
_RPS = _N // _NS
_CW = 16


def _sc_aggregate(feats, src2d, dst2d, with_counts):
    mesh = plsc.VectorSubcoreMesh(core_axis_name="c", subcore_axis_name="s",
                                  num_cores=_NC, num_subcores=_NS)
    out_type = [jax.ShapeDtypeStruct((_NC, _N, _D), jnp.float32)]
    scratch = [
        pltpu.VMEM_SHARED((_N, _D), jnp.float32),
        pltpu.VMEM((_WPW, _W), jnp.int32),
        pltpu.VMEM((_WPW, _W), jnp.int32),
        pltpu.VMEM((_W, _D), jnp.float32),
        pltpu.VMEM((_W, _D), jnp.float32),
        pltpu.SemaphoreType.DMA,
        pltpu.SemaphoreType.DMA,
    ]
    if with_counts:
        out_type.append(jax.ShapeDtypeStruct((_NC, _N, _CW), jnp.float32))
        scratch += [
            pltpu.VMEM_SHARED((_N, _CW), jnp.float32),
            pltpu.VMEM((_W, _CW), jnp.float32),
            pltpu.VMEM((_W, _CW), jnp.float32),
        ]

    def body(feats_hbm, src_hbm, dst_hbm, *refs):
        if with_counts:
            (acc_out, cnt_out, acc_sh, src_v, dst_v, rows0, rows1,
             sem0, sem1, cnt_sh, ones_v, zeros_v) = refs
        else:
            (acc_out, acc_sh, src_v, dst_v, rows0, rows1,
             sem0, sem1) = refs
        c = lax.axis_index("c")
        s = lax.axis_index("s")
        w = c * _NS + s

        pltpu.sync_copy(src_hbm.at[pl.ds(w * _WPW, _WPW)], src_v)
        pltpu.sync_copy(dst_hbm.at[pl.ds(w * _WPW, _WPW)], dst_v)

        zero16 = jnp.zeros((16,), jnp.float32)

        @pl.loop(0, _W)
        def _(r):
            @pl.loop(0, _D, step=16)
            def _(j):
                rows0[r, pl.ds(j, 16)] = zero16

        @pl.loop(0, _RPS // _W)
        def _(k):
            pltpu.sync_copy(rows0, acc_sh.at[pl.ds(s * _RPS + k * _W, _W)])

        if with_counts:
            one16 = jnp.ones((16,), jnp.float32)

            @pl.loop(0, _W)
            def _(r):
                ones_v[r, :] = one16
                zeros_v[r, :] = zero16

            @pl.loop(0, _RPS // _W)
            def _(k):
                pltpu.sync_copy(zeros_v,
                                cnt_sh.at[pl.ds(s * _RPS + k * _W, _W)])

        plsc.subcore_barrier()

        @pl.loop(0, _WPW, step=2)
        def _(i):
            g0 = pltpu.async_copy(feats_hbm.at[src_v.at[i]], rows0, sem0)
            g1 = pltpu.async_copy(feats_hbm.at[src_v.at[i + 1]], rows1, sem1)
            g0.wait()
            pltpu.sync_copy(rows0, acc_sh.at[dst_v.at[i]], add=True)
            if with_counts:
                pltpu.sync_copy(ones_v, cnt_sh.at[dst_v.at[i]], add=True)
            g1.wait()
            pltpu.sync_copy(rows1, acc_sh.at[dst_v.at[i + 1]], add=True)
            if with_counts:
                pltpu.sync_copy(ones_v, cnt_sh.at[dst_v.at[i + 1]], add=True)

        plsc.subcore_barrier()

        pltpu.sync_copy(acc_sh.at[pl.ds(s * _RPS, _RPS)],
                        acc_out.at[c, pl.ds(s * _RPS, _RPS)])
        if with_counts:
            pltpu.sync_copy(cnt_sh.at[pl.ds(s * _RPS, _RPS)],
                            cnt_out.at[c, pl.ds(s * _RPS, _RPS)])

    f = pl.kernel(body, out_type=out_type, mesh=mesh, scratch_types=scratch)
    return f(feats, src2d, dst2d)


def _tc_layer(acc, cnt, feats, w_l, w_r, b, do_relu):
    def body(acc_ref, cnt_ref, x_ref, wl_ref, wr_ref, b_ref, o_ref):
        total = acc_ref[0] + acc_ref[1]
        cnt = cnt_ref[0][:, 0:1] + cnt_ref[1][:, 0:1]
        mean = total / jnp.maximum(cnt, 1.0)
        y = (jnp.dot(mean, wl_ref[...], preferred_element_type=jnp.float32)
             + b_ref[...]
             + jnp.dot(x_ref[...], wr_ref[...],
                       preferred_element_type=jnp.float32))
        if do_relu:
            y = jnp.maximum(y, 0.0)
        o_ref[...] = y

    return pl.pallas_call(
        body,
        out_shape=jax.ShapeDtypeStruct((_N, _D), jnp.float32),
    )(acc, cnt, feats, w_l, w_r, b)


def kernel(x, edge_index, W1_l, W1_r, b1, W2_l, W2_r, b2):
    ei = edge_index.astype(jnp.int32)
    src2d = ei[0].reshape(_E // _W, _W)
    dst2d = ei[1].reshape(_E // _W, _W)
    acc1, cnt = _sc_aggregate(x, src2d, dst2d, True)
    h = _tc_layer(acc1, cnt, x, W1_l, W1_r, b1.reshape(1, _D), True)
    acc2 = _sc_aggregate(h, src2d, dst2d, False)[0]
    out = _tc_layer(acc2, cnt, h, W2_l, W2_r, b2.reshape(1, _D), False)
    return out


# speedup vs baseline: 11.1994x; 11.1994x over previous
import jax
import jax.numpy as jnp
from jax import lax
from jax.experimental import pallas as pl
from jax.experimental.pallas import tpu as pltpu
from jax.experimental.pallas import tpu_sc as plsc

_N = 10000
_D = 128
_E = 320000
_NC = 2
_NS = 16
_W = 125
_WPW = _E // (_NC * _NS * _W)
_MB = 8
_NMB = _WPW // _MB
_NP = 10240
_RPS = _NP // _NS
_ZCH = 80
_CW = 128


def _zero_rows(buf, nrows, ncols):
    zero16 = jnp.zeros((16,), jnp.float32)

    @pl.loop(0, nrows)
    def _(r):
        @pl.loop(0, ncols, step=16)
        def _(j):
            buf[r, pl.ds(j, 16)] = zero16


def _sc_aggregate(feats, src2d, dst2d):
    mesh = plsc.VectorSubcoreMesh(core_axis_name="c", subcore_axis_name="s",
                                  num_cores=_NC, num_subcores=_NS)
    out_type = jax.ShapeDtypeStruct((_NC, _NP, _D), jnp.float32)
    scratch = [
        pltpu.VMEM_SHARED((_NP, _D), jnp.float32),
        pltpu.VMEM((2, _MB, _W), jnp.int32),
        pltpu.VMEM((2, _MB, _W), jnp.int32),
        pltpu.VMEM((_W, _D), jnp.float32),
        pltpu.VMEM((_W, _D), jnp.float32),
        pltpu.SemaphoreType.DMA,
        pltpu.SemaphoreType.DMA,
        pltpu.SemaphoreType.DMA,
        pltpu.SemaphoreType.DMA,
    ]

    def body(feats_hbm, src_hbm, dst_hbm, acc_out, acc_sh,
             sbuf, dbuf, rows0, rows1, sem_g0, sem_g1, sem_i0, sem_i1):
        c = lax.axis_index("c")
        s = lax.axis_index("s")
        base = (c * _NS + s) * _WPW

        _zero_rows(rows0, _W, _D)

        @pl.loop(0, _RPS // _ZCH)
        def _(k):
            pltpu.sync_copy(rows0.at[pl.ds(0, _ZCH)],
                            acc_sh.at[pl.ds(s * _RPS + k * _ZCH, _ZCH)])

        plsc.subcore_barrier()

        rows = (rows0, rows1)
        gsem = (sem_g0, sem_g1)

        def load_idx(mb, slot, sem):
            pltpu.async_copy(src_hbm.at[pl.ds(base + mb * _MB, _MB)],
                             sbuf.at[slot], sem)
            pltpu.async_copy(dst_hbm.at[pl.ds(base + mb * _MB, _MB)],
                             dbuf.at[slot], sem)

        def wait_idx(slot, sem):
            pltpu.make_async_copy(src_hbm.at[pl.ds(0, _MB)],
                                  sbuf.at[slot], sem).wait()
            pltpu.make_async_copy(dst_hbm.at[pl.ds(0, _MB)],
                                  dbuf.at[slot], sem).wait()

        def gather(slot, j, r):
            return pltpu.async_copy(feats_hbm.at[sbuf.at[slot, j]],
                                    rows[r], gsem[r])

        def wait_gather(r):
            pltpu.make_async_copy(feats_hbm.at[sbuf.at[0, 0]],
                                  rows[r], gsem[r]).wait()

        def scatter(slot, j, r):
            pltpu.sync_copy(rows[r], acc_sh.at[dbuf.at[slot, j]], add=True)

        load_idx(0, 0, sem_i0)
        wait_idx(0, sem_i0)
        gather(0, 0, 0)
        load_idx(1, 1, sem_i1)

        @pl.loop(0, _NMB, step=2)
        def _(mb):
            for j in range(_MB):
                r = j % 2
                if j < _MB - 1:
                    gather(0, j + 1, 1 - r)
                else:
                    wait_idx(1, sem_i1)
                    gather(1, 0, 1 - r)
                wait_gather(r)
                scatter(0, j, r)

            @pl.when(mb + 2 < _NMB)
            def _():
                load_idx(mb + 2, 0, sem_i0)

            for j in range(_MB):
                r = j % 2
                if j < _MB - 1:
                    gather(1, j + 1, 1 - r)
                else:
                    @pl.when(mb + 2 < _NMB)
                    def _():
                        wait_idx(0, sem_i0)
                        gather(0, 0, 1 - r)
                wait_gather(r)
                scatter(1, j, r)

            @pl.when(mb + 3 < _NMB)
            def _():
                load_idx(mb + 3, 1, sem_i1)

        plsc.subcore_barrier()

        pltpu.sync_copy(acc_sh.at[pl.ds(s * _RPS, _RPS)],
                        acc_out.at[c, pl.ds(s * _RPS, _RPS)])

    f = pl.kernel(body, out_type=out_type, mesh=mesh, scratch_types=scratch)
    return f(feats, src2d, dst2d)


def _sc_counts(dst2d, ones_hbm, zeros_hbm):
    mesh = plsc.VectorSubcoreMesh(core_axis_name="c", subcore_axis_name="s",
                                  num_cores=_NC, num_subcores=_NS)
    out_type = jax.ShapeDtypeStruct((_NC, _NP, _CW), jnp.float32)
    scratch = [
        pltpu.VMEM_SHARED((_NP, _CW), jnp.float32),
        pltpu.VMEM((2, _MB, _W), jnp.int32),
        pltpu.VMEM((_W, _CW), jnp.float32),
        pltpu.SemaphoreType.DMA,
        pltpu.SemaphoreType.DMA,
    ]

    def body(dst_hbm, ones_hbm_ref, zeros_hbm_ref, cnt_out, cnt_sh, dbuf,
             ones_v, sem_i0, sem_i1):
        c = lax.axis_index("c")
        s = lax.axis_index("s")
        base = (c * _NS + s) * _WPW

        pltpu.sync_copy(ones_hbm_ref, ones_v)
        pltpu.sync_copy(zeros_hbm_ref, cnt_sh.at[pl.ds(s * _RPS, _RPS)])

        plsc.subcore_barrier()

        def load_idx(mb, slot, sem):
            pltpu.async_copy(dst_hbm.at[pl.ds(base + mb * _MB, _MB)],
                             dbuf.at[slot], sem)

        def wait_idx(slot, sem):
            pltpu.make_async_copy(dst_hbm.at[pl.ds(0, _MB)],
                                  dbuf.at[slot], sem).wait()

        load_idx(0, 0, sem_i0)
        wait_idx(0, sem_i0)
        load_idx(1, 1, sem_i1)

        @pl.loop(0, _NMB, step=2)
        def _(mb):
            for j in range(_MB):
                pltpu.sync_copy(ones_v, cnt_sh.at[dbuf.at[0, j]], add=True)

            @pl.when(mb + 2 < _NMB)
            def _():
                load_idx(mb + 2, 0, sem_i0)

            wait_idx(1, sem_i1)
            for j in range(_MB):
                pltpu.sync_copy(ones_v, cnt_sh.at[dbuf.at[1, j]], add=True)

            @pl.when(mb + 3 < _NMB)
            def _():
                load_idx(mb + 3, 1, sem_i1)

            @pl.when(mb + 2 < _NMB)
            def _():
                wait_idx(0, sem_i0)

        plsc.subcore_barrier()

        pltpu.sync_copy(cnt_sh.at[pl.ds(s * _RPS, _RPS)],
                        cnt_out.at[c, pl.ds(s * _RPS, _RPS)])

    f = pl.kernel(body, out_type=out_type, mesh=mesh, scratch_types=scratch)
    return f(dst2d, ones_hbm, zeros_hbm)


def _tc_layer(acc, cnt, feats, w_l, w_r, b, do_relu):
    def body(acc_ref, cnt_ref, x_ref, wl_ref, wr_ref, b_ref, o_ref):
        total = acc_ref[0][: _N] + acc_ref[1][: _N]
        cnt = cnt_ref[0][: _N, 0:1] + cnt_ref[1][: _N, 0:1]
        mean = total / jnp.maximum(cnt, 1.0)
        y = (jnp.dot(mean, wl_ref[...], preferred_element_type=jnp.float32)
             + b_ref[...]
             + jnp.dot(x_ref[...], wr_ref[...],
                       preferred_element_type=jnp.float32))
        if do_relu:
            y = jnp.maximum(y, 0.0)
        o_ref[...] = y

    return pl.pallas_call(
        body,
        out_shape=jax.ShapeDtypeStruct((_N, _D), jnp.float32),
    )(acc, cnt, feats, w_l, w_r, b)


def kernel(x, edge_index, W1_l, W1_r, b1, W2_l, W2_r, b2):
    ei = edge_index.astype(jnp.int32)
    src2d = ei[0].reshape(_E // _W, _W)
    dst2d = ei[1].reshape(_E // _W, _W)
    cnt = _sc_counts(dst2d, jnp.ones((_W, _CW), jnp.float32),
                     jnp.zeros((_RPS, _CW), jnp.float32))
    acc1 = _sc_aggregate(x, src2d, dst2d)
    h = _tc_layer(acc1, cnt, x, W1_l, W1_r, b1.reshape(1, _D), True)
    acc2 = _sc_aggregate(h, src2d, dst2d)
    out = _tc_layer(acc2, cnt, h, W2_l, W2_r, b2.reshape(1, _D), False)
    return out
